# baseline (speedup 1.0000x reference)
import functools

import jax
import jax.numpy as jnp
from jax import lax
from jax.experimental import pallas as pl
from jax.experimental.pallas import tpu as pltpu
from jax.experimental.pallas import tpu_sc as plsc

N = 10000
E = 320000
D = 128
H = 64

NC = 2
NS = 16
CH = 128
E2 = 327680
EPT = E2 // NS
NCHK = EPT // CH
G = 10
NSUP = NCHK // G
RPT = (2 * N) // NS
AROW = 2 * N + 8


def _lrelu(v):
    return jnp.where(v >= 0, v, 0.01 * v)


def _dot(a, b):
    return jnp.dot(a, b, preferred_element_type=jnp.float32)


BLK = 2000
GRID = N // BLK

_row = lambda i: (i, 0)
_fix = lambda i: (0, 0)


def _edges_body(src_ref, dst_ref, typ_ref, idxs_ref, sdx_ref, sdxb_ref):
    base = 2 * (src_ref[...] + typ_ref[...] * N)
    idxs_ref[0] = base
    idxs_ref[1] = base + 1
    sdx_ref[...] = 2 * dst_ref[...] + typ_ref[...]
    sdxb_ref[...] = dst_ref[...] + typ_ref[...] * N


_edges = pl.pallas_call(
    _edges_body,
    out_shape=[
        jax.ShapeDtypeStruct((2, E2 // D, D), jnp.int32),
        jax.ShapeDtypeStruct((E2 // D, D), jnp.int32),
        jax.ShapeDtypeStruct((E2 // D, D), jnp.int32),
    ],
)


def _prestage_body(nump_ref, catp_ref, wn_ref, bn_ref, wc_ref, bc_ref,
                   wi_ref, bi_ref, wr0_ref, wr1_ref, wroot_ref, brgcn_ref,
                   tab_ref, root_ref):
    n = _lrelu(_dot(nump_ref[...], wn_ref[...]) + bn_ref[...])
    c = _lrelu(_dot(catp_ref[...], wc_ref[...]) + bc_ref[...])
    x = jnp.concatenate((n, c), axis=1)
    x = _lrelu(_dot(x, wi_ref[...]) + bi_ref[...])
    tab_ref[0] = _dot(x, wr0_ref[...])
    tab_ref[1] = _dot(x, wr1_ref[...])
    root_ref[...] = _dot(x, wroot_ref[...]) + brgcn_ref[...]


_TAB_SPEC = pl.BlockSpec((2, BLK, D), lambda i: (0, i, 0))
_TAB_OUT = jax.ShapeDtypeStruct((2, N, D), jnp.float32)
_W_SPECS = [
    pl.BlockSpec((D, D), _fix),
    pl.BlockSpec((D, D), _fix),
    pl.BlockSpec((D, D), _fix),
    pl.BlockSpec((1, D), _fix),
]

_prestage = pl.pallas_call(
    _prestage_body,
    grid=(GRID,),
    in_specs=[
        pl.BlockSpec((BLK, 8), _row),
        pl.BlockSpec((BLK, 16), _row),
        pl.BlockSpec((8, H), _fix),
        pl.BlockSpec((1, H), _fix),
        pl.BlockSpec((16, H), _fix),
        pl.BlockSpec((1, H), _fix),
        pl.BlockSpec((D, D), _fix),
        pl.BlockSpec((1, D), _fix),
    ] + _W_SPECS,
    out_specs=[_TAB_SPEC, pl.BlockSpec((BLK, D), _row)],
    out_shape=[_TAB_OUT, jax.ShapeDtypeStruct((N, D), jnp.float32)],
)


def _combine(a0, a1, c00, c10, c01, c11, root):
    agg0 = jnp.concatenate((a0[:, 0:64], a1[:, 0:64]), axis=1)
    agg1 = jnp.concatenate((a0[:, 64:128], a1[:, 64:128]), axis=1)
    inv0 = 1.0 / jnp.maximum(c00[:, 0:1] + c10[:, 0:1], 1.0)
    inv1 = 1.0 / jnp.maximum(c01[:, 0:1] + c11[:, 0:1], 1.0)
    return root + agg0 * inv0 + agg1 * inv1


_ACC_SPECS = [
    pl.BlockSpec((1, BLK, D), lambda i: (0, i, 0)),
    pl.BlockSpec((1, BLK, D), lambda i: (1, i, 0)),
    pl.BlockSpec((1, BLK, 16), lambda i: (0, i, 0)),
    pl.BlockSpec((1, BLK, 16), lambda i: (1, i, 0)),
    pl.BlockSpec((1, BLK, 16), lambda i: (0, GRID + i, 0)),
    pl.BlockSpec((1, BLK, 16), lambda i: (1, GRID + i, 0)),
    pl.BlockSpec((BLK, D), _row),
]


def _mid_body(a0_ref, a1_ref, c00_ref, c10_ref, c01_ref, c11_ref, root_ref,
              wr0_ref, wr1_ref, wroot_ref, brgcn_ref, tab_ref, root2_ref):
    x1 = _combine(a0_ref[0], a1_ref[0], c00_ref[0], c10_ref[0],
                  c01_ref[0], c11_ref[0], root_ref[...])
    tab_ref[0] = _dot(x1, wr0_ref[...])
    tab_ref[1] = _dot(x1, wr1_ref[...])
    root2_ref[...] = _dot(x1, wroot_ref[...]) + brgcn_ref[...]


_mid = pl.pallas_call(
    _mid_body,
    grid=(GRID,),
    in_specs=_ACC_SPECS + _W_SPECS,
    out_specs=[_TAB_SPEC, pl.BlockSpec((BLK, D), _row)],
    out_shape=[_TAB_OUT, jax.ShapeDtypeStruct((N, D), jnp.float32)],
)


def _head_body(a0_ref, a1_ref, c00_ref, c10_ref, c01_ref, c11_ref, root_ref,
               wo1_ref, bo1_ref, wo2_ref, bo2_ref, out_ref):
    x2 = _combine(a0_ref[0], a1_ref[0], c00_ref[0], c10_ref[0],
                  c01_ref[0], c11_ref[0], root_ref[...])
    h = _lrelu(_dot(x2, wo1_ref[...]) + bo1_ref[...])
    out_ref[...] = _dot(h, wo2_ref[...]) + bo2_ref[...]


_head = pl.pallas_call(
    _head_body,
    grid=(GRID,),
    in_specs=_ACC_SPECS + [
        pl.BlockSpec((D, D), _fix),
        pl.BlockSpec((1, D), _fix),
        pl.BlockSpec((D, D), _fix),
        pl.BlockSpec((1, D), _fix),
    ],
    out_specs=[pl.BlockSpec((BLK, D), _row)],
    out_shape=[jax.ShapeDtypeStruct((N, D), jnp.float32)],
)


def _make_sc_layer(with_counts: bool):
    mesh = plsc.VectorSubcoreMesh(core_axis_name="c", subcore_axis_name="s",
                                  num_cores=NC, num_subcores=NS)
    out_type = [jax.ShapeDtypeStruct((NC, NS, RPT, 64), jnp.float32)]
    scratch = [
        pltpu.VMEM((G, CH), jnp.int32),
        pltpu.VMEM((G, CH), jnp.int32),
        pltpu.VMEM((CH, 64), jnp.float32),
        pltpu.VMEM((CH, 64), jnp.float32),
        pltpu.VMEM_SHARED((AROW, 64), jnp.float32),
        pltpu.SemaphoreType.DMA,
        pltpu.SemaphoreType.DMA,
        pltpu.SemaphoreType.DMA,
        pltpu.SemaphoreType.DMA,
    ]
    if with_counts:
        out_type.append(jax.ShapeDtypeStruct((NC, NS, RPT, 16), jnp.float32))
        scratch += [
            pltpu.VMEM((G, CH), jnp.int32),
            pltpu.VMEM((CH, 16), jnp.float32),
            pltpu.VMEM_SHARED((AROW, 16), jnp.float32),
        ]

    def body(*refs):
        if with_counts:
            (idxs, sdxh, sdxb, tab, z64, z16, onesh,
             acc_out, cnt_out,
             idx_v, sdx_v, buf0, buf1, acc_sh, sg0, sg1, ss0, ss1,
             sdxb_v, ones_v, cnt_sh) = refs
        else:
            (idxs, sdxh, tab, z64,
             acc_out,
             idx_v, sdx_v, buf0, buf1, acc_sh, sg0, sg1, ss0, ss1) = refs

        c = lax.axis_index("c")
        s = lax.axis_index("s")
        r0 = s * RPT

        pltpu.sync_copy(z64, acc_sh.at[pl.ds(r0, RPT)])
        if with_counts:
            pltpu.sync_copy(z16, cnt_sh.at[pl.ds(r0, RPT)])
            pltpu.sync_copy(onesh, ones_v)

        @pl.when(s == 0)
        def _zero_dump():
            pltpu.sync_copy(z64.at[pl.ds(0, 8)], acc_sh.at[pl.ds(2 * N, 8)])
            if with_counts:
                pltpu.sync_copy(z16.at[pl.ds(0, 8)],
                                cnt_sh.at[pl.ds(2 * N, 8)])
        plsc.subcore_barrier()

        bufs = (buf0, buf1)
        sg = (sg0, sg1)
        ss = (ss0, ss1)

        def edge_super(g, carry):
            row = s * NCHK + g * G
            pltpu.sync_copy(idxs.at[c, pl.ds(row, G)], idx_v)
            pltpu.sync_copy(sdxh.at[pl.ds(row, G)], sdx_v)
            gth = [None, None]
            sct = [None, None]
            gth[0] = pltpu.async_copy(tab.at[idx_v.at[0]], bufs[0], sg[0])
            for j in range(G):
                b = j % 2
                if j + 1 < G:
                    if sct[1 - b] is not None:
                        sct[1 - b].wait()
                    gth[1 - b] = pltpu.async_copy(tab.at[idx_v.at[j + 1]],
                                                  bufs[1 - b], sg[1 - b])
                gth[b].wait()
                sct[b] = pltpu.async_copy(bufs[b], acc_sh.at[sdx_v.at[j]],
                                          ss[b], add=True)
            sct[0].wait()
            sct[1].wait()
            if with_counts:
                @pl.when((g % NC) == c)
                def _counts():
                    pltpu.sync_copy(sdxb.at[pl.ds(row, G)], sdxb_v)
                    for j in range(G):
                        pltpu.sync_copy(ones_v, cnt_sh.at[sdxb_v.at[j]],
                                        add=True)
            return carry

        lax.fori_loop(0, NSUP, edge_super, 0)

        plsc.subcore_barrier()
        pltpu.sync_copy(acc_sh.at[pl.ds(r0, RPT)], acc_out.at[c, s])
        if with_counts:
            pltpu.sync_copy(cnt_sh.at[pl.ds(r0, RPT)], cnt_out.at[c, s])

    return pl.kernel(
        body, out_type=out_type, mesh=mesh, scratch_types=scratch,
        compiler_params=pltpu.CompilerParams(use_tc_tiling_on_sc=False))


@functools.lru_cache(maxsize=None)
def _sc_layers():
    return _make_sc_layer(with_counts=True), _make_sc_layer(with_counts=False)


def kernel(des, tweet, num_prop, cat_prop, edge_index, edge_type,
           Wn, bn, Wc, bc, Wi, bi, Wrel, Wroot, brgcn, Wo1, bo1, Wo2, bo2):
    del des, tweet

    nump = jnp.pad(num_prop, ((0, 0), (0, 2)))
    catp = jnp.pad(cat_prop, ((0, 0), (0, 5)))
    wn = jnp.pad(Wn, ((0, 2), (0, 0)))
    wc = jnp.pad(Wc, ((0, 5), (0, 0)))
    wo2 = jnp.pad(Wo2, ((0, 0), (0, D - 2)))
    bo2p = jnp.pad(bo2, (0, D - 2)).reshape(1, D)
    src = jnp.pad(edge_index[0], (0, E2 - E)).reshape(E2 // D, D)
    dst = jnp.pad(edge_index[1], (0, E2 - E),
                  constant_values=N).reshape(E2 // D, D)
    typ = jnp.pad(edge_type, (0, E2 - E),
                  constant_values=1).reshape(E2 // D, D)
    z64 = jnp.zeros((RPT, 64), jnp.float32)
    z16 = jnp.zeros((RPT, 16), jnp.float32)
    ones = jnp.ones((CH, 16), jnp.float32)

    idxs, sdx, sdxb = _edges(src, dst, typ)
    idxs4 = idxs.reshape(2, E2 // CH, CH)
    sdx3 = sdx.reshape(E2 // CH, CH)
    sdxb3 = sdxb.reshape(E2 // CH, CH)

    tab1, root1 = _prestage(
        nump, catp, wn, bn.reshape(1, H), wc, bc.reshape(1, H),
        Wi, bi.reshape(1, D), Wrel[0], Wrel[1], Wroot, brgcn.reshape(1, D))

    sc_layer1, sc_layer2 = _sc_layers()
    acc1, cnt = sc_layer1(idxs4, sdx3, sdxb3, tab1.reshape(4 * N, 64),
                          z64, z16, ones)
    acc1 = acc1.reshape(NC, N, D)
    cnt = cnt.reshape(NC, 2 * N, 16)

    tab2, root2 = _mid(acc1, acc1, cnt, cnt, cnt, cnt, root1,
                       Wrel[0], Wrel[1], Wroot, brgcn.reshape(1, D))

    (acc2,) = sc_layer2(idxs4, sdx3, tab2.reshape(4 * N, 64), z64)
    acc2 = acc2.reshape(NC, N, D)

    (outp,) = _head(acc2, acc2, cnt, cnt, cnt, cnt, root2,
                    Wo1, bo1.reshape(1, D), wo2, bo2p)
    return outp[:, 0:2]


# speedup vs baseline: 1.0010x; 1.0010x over previous
import functools

import jax
import jax.numpy as jnp
from jax import lax
from jax.experimental import pallas as pl
from jax.experimental.pallas import tpu as pltpu
from jax.experimental.pallas import tpu_sc as plsc

N = 10000
E = 320000
D = 128
H = 64

NC = 2
NS = 16
CH = 128
E2 = 327680
EPT = E2 // NS
NCHK = EPT // CH
G = 10
NSUP = NCHK // G
RPT = (2 * N) // NS
AROW = 2 * N + 8


def _lrelu(v):
    return jnp.where(v >= 0, v, 0.01 * v)


def _dot(a, b):
    return jnp.dot(a, b, preferred_element_type=jnp.float32)


BLK = 2000
GRID = N // BLK

_row = lambda i: (i, 0)
_fix = lambda i: (0, 0)


def _edges_body(src_ref, dst_ref, typ_ref, idxs_ref, sdx_ref, sdxb_ref):
    base = 2 * (src_ref[...] + typ_ref[...] * N)
    idxs_ref[0] = base
    idxs_ref[1] = base + 1
    sdx_ref[...] = 2 * dst_ref[...] + typ_ref[...]
    sdxb_ref[...] = dst_ref[...] + typ_ref[...] * N


_edges = pl.pallas_call(
    _edges_body,
    out_shape=[
        jax.ShapeDtypeStruct((2, E2 // D, D), jnp.int32),
        jax.ShapeDtypeStruct((E2 // D, D), jnp.int32),
        jax.ShapeDtypeStruct((E2 // D, D), jnp.int32),
    ],
)


def _prestage_body(nump_ref, catp_ref, wn_ref, bn_ref, wc_ref, bc_ref,
                   wi_ref, bi_ref, wr0_ref, wr1_ref, wroot_ref, brgcn_ref,
                   tab_ref, root_ref):
    n = _lrelu(_dot(nump_ref[...], wn_ref[...]) + bn_ref[...])
    c = _lrelu(_dot(catp_ref[...], wc_ref[...]) + bc_ref[...])
    x = jnp.concatenate((n, c), axis=1)
    x = _lrelu(_dot(x, wi_ref[...]) + bi_ref[...])
    tab_ref[0] = _dot(x, wr0_ref[...])
    tab_ref[1] = _dot(x, wr1_ref[...])
    root_ref[...] = _dot(x, wroot_ref[...]) + brgcn_ref[...]


_TAB_SPEC = pl.BlockSpec((2, BLK, D), lambda i: (0, i, 0))
_TAB_OUT = jax.ShapeDtypeStruct((2, N, D), jnp.float32)
_W_SPECS = [
    pl.BlockSpec((D, D), _fix),
    pl.BlockSpec((D, D), _fix),
    pl.BlockSpec((D, D), _fix),
    pl.BlockSpec((1, D), _fix),
]

_prestage = pl.pallas_call(
    _prestage_body,
    grid=(GRID,),
    in_specs=[
        pl.BlockSpec((BLK, 8), _row),
        pl.BlockSpec((BLK, 16), _row),
        pl.BlockSpec((8, H), _fix),
        pl.BlockSpec((1, H), _fix),
        pl.BlockSpec((16, H), _fix),
        pl.BlockSpec((1, H), _fix),
        pl.BlockSpec((D, D), _fix),
        pl.BlockSpec((1, D), _fix),
    ] + _W_SPECS,
    out_specs=[_TAB_SPEC, pl.BlockSpec((BLK, D), _row)],
    out_shape=[_TAB_OUT, jax.ShapeDtypeStruct((N, D), jnp.float32)],
)


def _combine(a0, a1, c00, c10, c01, c11, root):
    agg0 = jnp.concatenate((a0[:, 0:64], a1[:, 0:64]), axis=1)
    agg1 = jnp.concatenate((a0[:, 64:128], a1[:, 64:128]), axis=1)
    inv0 = 1.0 / jnp.maximum(c00[:, 0:1] + c10[:, 0:1], 1.0)
    inv1 = 1.0 / jnp.maximum(c01[:, 0:1] + c11[:, 0:1], 1.0)
    return root + agg0 * inv0 + agg1 * inv1


_ACC_SPECS = [
    pl.BlockSpec((1, BLK, D), lambda i: (0, i, 0)),
    pl.BlockSpec((1, BLK, D), lambda i: (1, i, 0)),
    pl.BlockSpec((1, BLK, 16), lambda i: (0, i, 0)),
    pl.BlockSpec((1, BLK, 16), lambda i: (1, i, 0)),
    pl.BlockSpec((1, BLK, 16), lambda i: (0, GRID + i, 0)),
    pl.BlockSpec((1, BLK, 16), lambda i: (1, GRID + i, 0)),
    pl.BlockSpec((BLK, D), _row),
]


def _mid_body(a0_ref, a1_ref, c00_ref, c10_ref, c01_ref, c11_ref, root_ref,
              wr0_ref, wr1_ref, wroot_ref, brgcn_ref, tab_ref, root2_ref):
    x1 = _combine(a0_ref[0], a1_ref[0], c00_ref[0], c10_ref[0],
                  c01_ref[0], c11_ref[0], root_ref[...])
    tab_ref[0] = _dot(x1, wr0_ref[...])
    tab_ref[1] = _dot(x1, wr1_ref[...])
    root2_ref[...] = _dot(x1, wroot_ref[...]) + brgcn_ref[...]


_mid = pl.pallas_call(
    _mid_body,
    grid=(GRID,),
    in_specs=_ACC_SPECS + _W_SPECS,
    out_specs=[_TAB_SPEC, pl.BlockSpec((BLK, D), _row)],
    out_shape=[_TAB_OUT, jax.ShapeDtypeStruct((N, D), jnp.float32)],
)


def _head_body(a0_ref, a1_ref, c00_ref, c10_ref, c01_ref, c11_ref, root_ref,
               wo1_ref, bo1_ref, wo2_ref, bo2_ref, out_ref):
    x2 = _combine(a0_ref[0], a1_ref[0], c00_ref[0], c10_ref[0],
                  c01_ref[0], c11_ref[0], root_ref[...])
    h = _lrelu(_dot(x2, wo1_ref[...]) + bo1_ref[...])
    out_ref[...] = _dot(h, wo2_ref[...]) + bo2_ref[...]


_head = pl.pallas_call(
    _head_body,
    grid=(GRID,),
    in_specs=_ACC_SPECS + [
        pl.BlockSpec((D, D), _fix),
        pl.BlockSpec((1, D), _fix),
        pl.BlockSpec((D, D), _fix),
        pl.BlockSpec((1, D), _fix),
    ],
    out_specs=[pl.BlockSpec((BLK, D), _row)],
    out_shape=[jax.ShapeDtypeStruct((N, D), jnp.float32)],
)


def _make_sc_layer(with_counts: bool):
    mesh = plsc.VectorSubcoreMesh(core_axis_name="c", subcore_axis_name="s",
                                  num_cores=NC, num_subcores=NS)
    out_type = [jax.ShapeDtypeStruct((NC, NS, RPT, 64), jnp.float32)]
    scratch = [
        pltpu.VMEM((G, CH), jnp.int32),
        pltpu.VMEM((G, CH), jnp.int32),
        pltpu.VMEM((CH, 64), jnp.float32),
        pltpu.VMEM((CH, 64), jnp.float32),
        pltpu.VMEM_SHARED((AROW, 64), jnp.float32),
        pltpu.SemaphoreType.DMA,
        pltpu.SemaphoreType.DMA,
        pltpu.SemaphoreType.DMA,
        pltpu.SemaphoreType.DMA,
    ]
    if with_counts:
        out_type.append(jax.ShapeDtypeStruct((NC, NS, RPT, 16), jnp.float32))
        scratch += [
            pltpu.VMEM((G, CH), jnp.int32),
            pltpu.VMEM((CH, 16), jnp.float32),
            pltpu.VMEM_SHARED((AROW, 16), jnp.float32),
        ]

    def body(*refs):
        if with_counts:
            (idxs, sdxh, sdxb, tab, z64, z16, onesh,
             acc_out, cnt_out,
             idx_v, sdx_v, buf0, buf1, acc_sh, sg0, sg1, ss0, ss1,
             sdxb_v, ones_v, cnt_sh) = refs
        else:
            (idxs, sdxh, tab, z64,
             acc_out,
             idx_v, sdx_v, buf0, buf1, acc_sh, sg0, sg1, ss0, ss1) = refs

        c = lax.axis_index("c")
        s = lax.axis_index("s")
        r0 = s * RPT

        pltpu.sync_copy(z64, acc_sh.at[pl.ds(r0, RPT)])
        if with_counts:
            pltpu.sync_copy(z16, cnt_sh.at[pl.ds(r0, RPT)])
            pltpu.sync_copy(onesh, ones_v)

        @pl.when(s == 0)
        def _zero_dump():
            pltpu.sync_copy(z64.at[pl.ds(0, 8)], acc_sh.at[pl.ds(2 * N, 8)])
            if with_counts:
                pltpu.sync_copy(z16.at[pl.ds(0, 8)],
                                cnt_sh.at[pl.ds(2 * N, 8)])
        plsc.subcore_barrier()

        bufs = (buf0, buf1)
        sg = (sg0, sg1)
        ss = (ss0, ss1)

        def edge_super(g, carry):
            row = s * NCHK + g * G
            pltpu.sync_copy(idxs.at[c, pl.ds(row, G)], idx_v)
            pltpu.sync_copy(sdxh.at[pl.ds(row, G)], sdx_v)
            gth = [None, None]
            gth[0] = pltpu.async_copy(tab.at[idx_v.at[0]], bufs[0], sg[0])
            for j in range(G):
                b = j % 2
                if j + 1 < G:
                    gth[1 - b] = pltpu.async_copy(tab.at[idx_v.at[j + 1]],
                                                  bufs[1 - b], sg[1 - b])
                gth[b].wait()
                pltpu.sync_copy(bufs[b], acc_sh.at[sdx_v.at[j]], add=True)
            if with_counts:
                @pl.when((g % NC) == c)
                def _counts():
                    pltpu.sync_copy(sdxb.at[pl.ds(row, G)], sdxb_v)
                    for j in range(G):
                        pltpu.sync_copy(ones_v, cnt_sh.at[sdxb_v.at[j]],
                                        add=True)
            return carry

        lax.fori_loop(0, NSUP, edge_super, 0)

        plsc.subcore_barrier()
        pltpu.sync_copy(acc_sh.at[pl.ds(r0, RPT)], acc_out.at[c, s])
        if with_counts:
            pltpu.sync_copy(cnt_sh.at[pl.ds(r0, RPT)], cnt_out.at[c, s])

    return pl.kernel(
        body, out_type=out_type, mesh=mesh, scratch_types=scratch,
        compiler_params=pltpu.CompilerParams(use_tc_tiling_on_sc=False))


@functools.lru_cache(maxsize=None)
def _sc_layers():
    return _make_sc_layer(with_counts=True), _make_sc_layer(with_counts=False)


def kernel(des, tweet, num_prop, cat_prop, edge_index, edge_type,
           Wn, bn, Wc, bc, Wi, bi, Wrel, Wroot, brgcn, Wo1, bo1, Wo2, bo2):
    del des, tweet

    nump = jnp.pad(num_prop, ((0, 0), (0, 2)))
    catp = jnp.pad(cat_prop, ((0, 0), (0, 5)))
    wn = jnp.pad(Wn, ((0, 2), (0, 0)))
    wc = jnp.pad(Wc, ((0, 5), (0, 0)))
    wo2 = jnp.pad(Wo2, ((0, 0), (0, D - 2)))
    bo2p = jnp.pad(bo2, (0, D - 2)).reshape(1, D)
    src = jnp.pad(edge_index[0], (0, E2 - E)).reshape(E2 // D, D)
    dst = jnp.pad(edge_index[1], (0, E2 - E),
                  constant_values=N).reshape(E2 // D, D)
    typ = jnp.pad(edge_type, (0, E2 - E),
                  constant_values=1).reshape(E2 // D, D)
    z64 = jnp.zeros((RPT, 64), jnp.float32)
    z16 = jnp.zeros((RPT, 16), jnp.float32)
    ones = jnp.ones((CH, 16), jnp.float32)

    idxs, sdx, sdxb = _edges(src, dst, typ)
    idxs4 = idxs.reshape(2, E2 // CH, CH)
    sdx3 = sdx.reshape(E2 // CH, CH)
    sdxb3 = sdxb.reshape(E2 // CH, CH)

    tab1, root1 = _prestage(
        nump, catp, wn, bn.reshape(1, H), wc, bc.reshape(1, H),
        Wi, bi.reshape(1, D), Wrel[0], Wrel[1], Wroot, brgcn.reshape(1, D))

    sc_layer1, sc_layer2 = _sc_layers()
    acc1, cnt = sc_layer1(idxs4, sdx3, sdxb3, tab1.reshape(4 * N, 64),
                          z64, z16, ones)
    acc1 = acc1.reshape(NC, N, D)
    cnt = cnt.reshape(NC, 2 * N, 16)

    tab2, root2 = _mid(acc1, acc1, cnt, cnt, cnt, cnt, root1,
                       Wrel[0], Wrel[1], Wroot, brgcn.reshape(1, D))

    (acc2,) = sc_layer2(idxs4, sdx3, tab2.reshape(4 * N, 64), z64)
    acc2 = acc2.reshape(NC, N, D)

    (outp,) = _head(acc2, acc2, cnt, cnt, cnt, cnt, root2,
                    Wo1, bo1.reshape(1, D), wo2, bo2p)
    return outp[:, 0:2]


# speedup vs baseline: 1.8529x; 1.8511x over previous
import functools

import jax
import jax.numpy as jnp
from jax import lax
from jax.experimental import pallas as pl
from jax.experimental.pallas import tpu as pltpu
from jax.experimental.pallas import tpu_sc as plsc

N = 10000
E = 320000
D = 128
H = 64

NC = 2
NS = 16
CH = 80
E2 = E
EPT = E2 // NS
NCHK = EPT // CH
G = 10
NSUP = NCHK // G
RPT = (2 * N) // NS
AROW = 2 * N


def _lrelu(v):
    return jnp.where(v >= 0, v, 0.01 * v)


def _dot(a, b):
    return jnp.dot(a, b, preferred_element_type=jnp.float32)


BLK = 2000
GRID = N // BLK

_row = lambda i: (i, 0)
_fix = lambda i: (0, 0)


def _edges_body(src_ref, dst_ref, typ_ref, idxs_ref, sdx_ref, sdxb_ref):
    base = 2 * (src_ref[...] + typ_ref[...] * N)
    idxs_ref[0] = base
    idxs_ref[1] = base + 1
    sdx_ref[...] = 2 * dst_ref[...] + typ_ref[...]
    sdxb_ref[...] = dst_ref[...] + typ_ref[...] * N


_edges = pl.pallas_call(
    _edges_body,
    out_shape=[
        jax.ShapeDtypeStruct((2, E2 // D, D), jnp.int32),
        jax.ShapeDtypeStruct((E2 // D, D), jnp.int32),
        jax.ShapeDtypeStruct((E2 // D, D), jnp.int32),
    ],
)


def _prestage_body(nump_ref, catp_ref, wn_ref, bn_ref, wc_ref, bc_ref,
                   wi_ref, bi_ref, wr0_ref, wr1_ref, wroot_ref, brgcn_ref,
                   tab_ref, root_ref):
    n = _lrelu(_dot(nump_ref[...], wn_ref[...]) + bn_ref[...])
    c = _lrelu(_dot(catp_ref[...], wc_ref[...]) + bc_ref[...])
    x = jnp.concatenate((n, c), axis=1)
    x = _lrelu(_dot(x, wi_ref[...]) + bi_ref[...])
    tab_ref[0] = _dot(x, wr0_ref[...])
    tab_ref[1] = _dot(x, wr1_ref[...])
    root_ref[...] = _dot(x, wroot_ref[...]) + brgcn_ref[...]


_TAB_SPEC = pl.BlockSpec((2, BLK, D), lambda i: (0, i, 0))
_TAB_OUT = jax.ShapeDtypeStruct((2, N, D), jnp.float32)
_W_SPECS = [
    pl.BlockSpec((D, D), _fix),
    pl.BlockSpec((D, D), _fix),
    pl.BlockSpec((D, D), _fix),
    pl.BlockSpec((1, D), _fix),
]

_prestage = pl.pallas_call(
    _prestage_body,
    grid=(GRID,),
    in_specs=[
        pl.BlockSpec((BLK, 8), _row),
        pl.BlockSpec((BLK, 16), _row),
        pl.BlockSpec((8, H), _fix),
        pl.BlockSpec((1, H), _fix),
        pl.BlockSpec((16, H), _fix),
        pl.BlockSpec((1, H), _fix),
        pl.BlockSpec((D, D), _fix),
        pl.BlockSpec((1, D), _fix),
    ] + _W_SPECS,
    out_specs=[_TAB_SPEC, pl.BlockSpec((BLK, D), _row)],
    out_shape=[_TAB_OUT, jax.ShapeDtypeStruct((N, D), jnp.float32)],
)


def _combine(a0, a1, c00, c10, c01, c11, root):
    agg0 = jnp.concatenate((a0[:, 0:64], a1[:, 0:64]), axis=1)
    agg1 = jnp.concatenate((a0[:, 64:128], a1[:, 64:128]), axis=1)
    inv0 = 1.0 / jnp.maximum(c00[:, 0:1] + c10[:, 0:1], 1.0)
    inv1 = 1.0 / jnp.maximum(c01[:, 0:1] + c11[:, 0:1], 1.0)
    return root + agg0 * inv0 + agg1 * inv1


_ACC_SPECS = [
    pl.BlockSpec((1, BLK, D), lambda i: (0, i, 0)),
    pl.BlockSpec((1, BLK, D), lambda i: (1, i, 0)),
    pl.BlockSpec((1, BLK, 16), lambda i: (0, i, 0)),
    pl.BlockSpec((1, BLK, 16), lambda i: (1, i, 0)),
    pl.BlockSpec((1, BLK, 16), lambda i: (0, GRID + i, 0)),
    pl.BlockSpec((1, BLK, 16), lambda i: (1, GRID + i, 0)),
    pl.BlockSpec((BLK, D), _row),
]


def _mid_body(a0_ref, a1_ref, c00_ref, c10_ref, c01_ref, c11_ref, root_ref,
              wr0_ref, wr1_ref, wroot_ref, brgcn_ref, tab_ref, root2_ref):
    x1 = _combine(a0_ref[0], a1_ref[0], c00_ref[0], c10_ref[0],
                  c01_ref[0], c11_ref[0], root_ref[...])
    tab_ref[0] = _dot(x1, wr0_ref[...])
    tab_ref[1] = _dot(x1, wr1_ref[...])
    root2_ref[...] = _dot(x1, wroot_ref[...]) + brgcn_ref[...]


_mid = pl.pallas_call(
    _mid_body,
    grid=(GRID,),
    in_specs=_ACC_SPECS + _W_SPECS,
    out_specs=[_TAB_SPEC, pl.BlockSpec((BLK, D), _row)],
    out_shape=[_TAB_OUT, jax.ShapeDtypeStruct((N, D), jnp.float32)],
)


def _head_body(a0_ref, a1_ref, c00_ref, c10_ref, c01_ref, c11_ref, root_ref,
               wo1_ref, bo1_ref, wo2_ref, bo2_ref, out_ref):
    x2 = _combine(a0_ref[0], a1_ref[0], c00_ref[0], c10_ref[0],
                  c01_ref[0], c11_ref[0], root_ref[...])
    h = _lrelu(_dot(x2, wo1_ref[...]) + bo1_ref[...])
    out_ref[...] = _dot(h, wo2_ref[...]) + bo2_ref[...]


_head = pl.pallas_call(
    _head_body,
    grid=(GRID,),
    in_specs=_ACC_SPECS + [
        pl.BlockSpec((D, D), _fix),
        pl.BlockSpec((1, D), _fix),
        pl.BlockSpec((D, D), _fix),
        pl.BlockSpec((1, D), _fix),
    ],
    out_specs=[pl.BlockSpec((BLK, D), _row)],
    out_shape=[jax.ShapeDtypeStruct((N, D), jnp.float32)],
)


def _make_sc_layer(with_counts: bool):
    mesh = plsc.VectorSubcoreMesh(core_axis_name="c", subcore_axis_name="s",
                                  num_cores=NC, num_subcores=NS)
    out_type = [jax.ShapeDtypeStruct((NC, NS, RPT, 64), jnp.float32)]
    scratch = [
        pltpu.VMEM((G, CH), jnp.int32),
        pltpu.VMEM((G, CH), jnp.int32),
        pltpu.VMEM((CH, 64), jnp.float32),
        pltpu.VMEM((CH, 64), jnp.float32),
        pltpu.VMEM_SHARED((AROW, 64), jnp.float32),
        pltpu.SemaphoreType.DMA,
        pltpu.SemaphoreType.DMA,
        pltpu.SemaphoreType.DMA,
        pltpu.SemaphoreType.DMA,
    ]
    if with_counts:
        out_type.append(jax.ShapeDtypeStruct((NC, NS, RPT, 16), jnp.float32))
        scratch += [
            pltpu.VMEM((G, CH), jnp.int32),
            pltpu.VMEM((CH, 16), jnp.float32),
            pltpu.VMEM_SHARED((AROW, 16), jnp.float32),
        ]

    def body(*refs):
        if with_counts:
            (idxs, sdxh, sdxb, tab, z64, z16, onesh,
             acc_out, cnt_out,
             idx_v, sdx_v, buf0, buf1, acc_sh, sg0, sg1, ss0, ss1,
             sdxb_v, ones_v, cnt_sh) = refs
        else:
            (idxs, sdxh, tab, z64,
             acc_out,
             idx_v, sdx_v, buf0, buf1, acc_sh, sg0, sg1, ss0, ss1) = refs

        c = lax.axis_index("c")
        s = lax.axis_index("s")
        r0 = s * RPT

        pltpu.sync_copy(z64, acc_sh.at[pl.ds(r0, RPT)])
        if with_counts:
            pltpu.sync_copy(z16, cnt_sh.at[pl.ds(r0, RPT)])
            pltpu.sync_copy(onesh, ones_v)
        plsc.subcore_barrier()

        bufs = (buf0, buf1)
        sg = (sg0, sg1)
        ss = (ss0, ss1)

        def edge_super(g, carry):
            row = s * NCHK + g * G
            pltpu.sync_copy(idxs.at[c, pl.ds(row, G)], idx_v)
            pltpu.sync_copy(sdxh.at[pl.ds(row, G)], sdx_v)
            gth = [None, None]
            sct = [None, None]
            gth[0] = pltpu.async_copy(tab.at[idx_v.at[0]], bufs[0], sg[0])
            for j in range(G):
                b = j % 2
                if j + 1 < G:
                    if sct[1 - b] is not None:
                        sct[1 - b].wait()
                    gth[1 - b] = pltpu.async_copy(tab.at[idx_v.at[j + 1]],
                                                  bufs[1 - b], sg[1 - b])
                gth[b].wait()
                sct[b] = pltpu.async_copy(bufs[b], acc_sh.at[sdx_v.at[j]],
                                          ss[b], add=True)
            sct[0].wait()
            sct[1].wait()
            if with_counts:
                @pl.when((g % NC) == c)
                def _counts():
                    pltpu.sync_copy(sdxb.at[pl.ds(row, G)], sdxb_v)
                    for j in range(G):
                        pltpu.sync_copy(ones_v, cnt_sh.at[sdxb_v.at[j]],
                                        add=True)
            return carry

        lax.fori_loop(0, NSUP, edge_super, 0)

        plsc.subcore_barrier()
        pltpu.sync_copy(acc_sh.at[pl.ds(r0, RPT)], acc_out.at[c, s])
        if with_counts:
            pltpu.sync_copy(cnt_sh.at[pl.ds(r0, RPT)], cnt_out.at[c, s])

    return pl.kernel(
        body, out_type=out_type, mesh=mesh, scratch_types=scratch,
        compiler_params=pltpu.CompilerParams(use_tc_tiling_on_sc=False))


@functools.lru_cache(maxsize=None)
def _sc_layers():
    return _make_sc_layer(with_counts=True), _make_sc_layer(with_counts=False)


def kernel(des, tweet, num_prop, cat_prop, edge_index, edge_type,
           Wn, bn, Wc, bc, Wi, bi, Wrel, Wroot, brgcn, Wo1, bo1, Wo2, bo2):
    del des, tweet

    nump = jnp.pad(num_prop, ((0, 0), (0, 2)))
    catp = jnp.pad(cat_prop, ((0, 0), (0, 5)))
    wn = jnp.pad(Wn, ((0, 2), (0, 0)))
    wc = jnp.pad(Wc, ((0, 5), (0, 0)))
    wo2 = jnp.pad(Wo2, ((0, 0), (0, D - 2)))
    bo2p = jnp.pad(bo2, (0, D - 2)).reshape(1, D)
    src = edge_index[0].reshape(E2 // D, D)
    dst = edge_index[1].reshape(E2 // D, D)
    typ = edge_type.reshape(E2 // D, D)
    z64 = jnp.zeros((RPT, 64), jnp.float32)
    z16 = jnp.zeros((RPT, 16), jnp.float32)
    ones = jnp.ones((CH, 16), jnp.float32)

    idxs, sdx, sdxb = _edges(src, dst, typ)
    idxs4 = idxs.reshape(2, E2 // CH, CH)
    sdx3 = sdx.reshape(E2 // CH, CH)
    sdxb3 = sdxb.reshape(E2 // CH, CH)

    tab1, root1 = _prestage(
        nump, catp, wn, bn.reshape(1, H), wc, bc.reshape(1, H),
        Wi, bi.reshape(1, D), Wrel[0], Wrel[1], Wroot, brgcn.reshape(1, D))

    sc_layer1, sc_layer2 = _sc_layers()
    acc1, cnt = sc_layer1(idxs4, sdx3, sdxb3, tab1.reshape(4 * N, 64),
                          z64, z16, ones)
    acc1 = acc1.reshape(NC, N, D)
    cnt = cnt.reshape(NC, 2 * N, 16)

    tab2, root2 = _mid(acc1, acc1, cnt, cnt, cnt, cnt, root1,
                       Wrel[0], Wrel[1], Wroot, brgcn.reshape(1, D))

    (acc2,) = sc_layer2(idxs4, sdx3, tab2.reshape(4 * N, 64), z64)
    acc2 = acc2.reshape(NC, N, D)

    (outp,) = _head(acc2, acc2, cnt, cnt, cnt, cnt, root2,
                    Wo1, bo1.reshape(1, D), wo2, bo2p)
    return outp[:, 0:2]


# speedup vs baseline: 2.0108x; 1.0852x over previous
import functools

import jax
import jax.numpy as jnp
from jax import lax
from jax.experimental import pallas as pl
from jax.experimental.pallas import tpu as pltpu
from jax.experimental.pallas import tpu_sc as plsc

N = 10000
E = 320000
D = 128
H = 64

NC = 2
NS = 16
CH = 80
E2 = E
EPT = E2 // NS
NCHK = EPT // CH
G = 25
NSUP = NCHK // G
RPT = (2 * N) // NS
AROW = 2 * N


def _lrelu(v):
    return jnp.where(v >= 0, v, 0.01 * v)


def _dot(a, b):
    return jnp.dot(a, b, preferred_element_type=jnp.float32)


BLK = 2000
GRID = N // BLK

_row = lambda i: (i, 0)
_fix = lambda i: (0, 0)


def _edges_body(src_ref, dst_ref, typ_ref, idxs_ref, sdx_ref, sdxb_ref):
    base = 2 * (src_ref[...] + typ_ref[...] * N)
    idxs_ref[0] = base
    idxs_ref[1] = base + 1
    sdx_ref[...] = 2 * dst_ref[...] + typ_ref[...]
    sdxb_ref[...] = dst_ref[...] + typ_ref[...] * N


_edges = pl.pallas_call(
    _edges_body,
    out_shape=[
        jax.ShapeDtypeStruct((2, E2 // D, D), jnp.int32),
        jax.ShapeDtypeStruct((E2 // D, D), jnp.int32),
        jax.ShapeDtypeStruct((E2 // D, D), jnp.int32),
    ],
)


def _prestage_body(nump_ref, catp_ref, wn_ref, bn_ref, wc_ref, bc_ref,
                   wi_ref, bi_ref, wr0_ref, wr1_ref, wroot_ref, brgcn_ref,
                   tab_ref, root_ref):
    n = _lrelu(_dot(nump_ref[...], wn_ref[...]) + bn_ref[...])
    c = _lrelu(_dot(catp_ref[...], wc_ref[...]) + bc_ref[...])
    x = jnp.concatenate((n, c), axis=1)
    x = _lrelu(_dot(x, wi_ref[...]) + bi_ref[...])
    tab_ref[0] = _dot(x, wr0_ref[...])
    tab_ref[1] = _dot(x, wr1_ref[...])
    root_ref[...] = _dot(x, wroot_ref[...]) + brgcn_ref[...]


_TAB_SPEC = pl.BlockSpec((2, BLK, D), lambda i: (0, i, 0))
_TAB_OUT = jax.ShapeDtypeStruct((2, N, D), jnp.float32)
_W_SPECS = [
    pl.BlockSpec((D, D), _fix),
    pl.BlockSpec((D, D), _fix),
    pl.BlockSpec((D, D), _fix),
    pl.BlockSpec((1, D), _fix),
]

_prestage = pl.pallas_call(
    _prestage_body,
    grid=(GRID,),
    in_specs=[
        pl.BlockSpec((BLK, 8), _row),
        pl.BlockSpec((BLK, 16), _row),
        pl.BlockSpec((8, H), _fix),
        pl.BlockSpec((1, H), _fix),
        pl.BlockSpec((16, H), _fix),
        pl.BlockSpec((1, H), _fix),
        pl.BlockSpec((D, D), _fix),
        pl.BlockSpec((1, D), _fix),
    ] + _W_SPECS,
    out_specs=[_TAB_SPEC, pl.BlockSpec((BLK, D), _row)],
    out_shape=[_TAB_OUT, jax.ShapeDtypeStruct((N, D), jnp.float32)],
)


def _combine(a0, a1, c00, c10, c01, c11, root):
    agg0 = jnp.concatenate((a0[:, 0:64], a1[:, 0:64]), axis=1)
    agg1 = jnp.concatenate((a0[:, 64:128], a1[:, 64:128]), axis=1)
    inv0 = 1.0 / jnp.maximum(c00[:, 0:1] + c10[:, 0:1], 1.0)
    inv1 = 1.0 / jnp.maximum(c01[:, 0:1] + c11[:, 0:1], 1.0)
    return root + agg0 * inv0 + agg1 * inv1


_ACC_SPECS = [
    pl.BlockSpec((1, BLK, D), lambda i: (0, i, 0)),
    pl.BlockSpec((1, BLK, D), lambda i: (1, i, 0)),
    pl.BlockSpec((1, BLK, 16), lambda i: (0, i, 0)),
    pl.BlockSpec((1, BLK, 16), lambda i: (1, i, 0)),
    pl.BlockSpec((1, BLK, 16), lambda i: (0, GRID + i, 0)),
    pl.BlockSpec((1, BLK, 16), lambda i: (1, GRID + i, 0)),
    pl.BlockSpec((BLK, D), _row),
]


def _mid_body(a0_ref, a1_ref, c00_ref, c10_ref, c01_ref, c11_ref, root_ref,
              wr0_ref, wr1_ref, wroot_ref, brgcn_ref, tab_ref, root2_ref):
    x1 = _combine(a0_ref[0], a1_ref[0], c00_ref[0], c10_ref[0],
                  c01_ref[0], c11_ref[0], root_ref[...])
    tab_ref[0] = _dot(x1, wr0_ref[...])
    tab_ref[1] = _dot(x1, wr1_ref[...])
    root2_ref[...] = _dot(x1, wroot_ref[...]) + brgcn_ref[...]


_mid = pl.pallas_call(
    _mid_body,
    grid=(GRID,),
    in_specs=_ACC_SPECS + _W_SPECS,
    out_specs=[_TAB_SPEC, pl.BlockSpec((BLK, D), _row)],
    out_shape=[_TAB_OUT, jax.ShapeDtypeStruct((N, D), jnp.float32)],
)


def _head_body(a0_ref, a1_ref, c00_ref, c10_ref, c01_ref, c11_ref, root_ref,
               wo1_ref, bo1_ref, wo2_ref, bo2_ref, out_ref):
    x2 = _combine(a0_ref[0], a1_ref[0], c00_ref[0], c10_ref[0],
                  c01_ref[0], c11_ref[0], root_ref[...])
    h = _lrelu(_dot(x2, wo1_ref[...]) + bo1_ref[...])
    out_ref[...] = _dot(h, wo2_ref[...]) + bo2_ref[...]


_head = pl.pallas_call(
    _head_body,
    grid=(GRID,),
    in_specs=_ACC_SPECS + [
        pl.BlockSpec((D, D), _fix),
        pl.BlockSpec((1, D), _fix),
        pl.BlockSpec((D, D), _fix),
        pl.BlockSpec((1, D), _fix),
    ],
    out_specs=[pl.BlockSpec((BLK, D), _row)],
    out_shape=[jax.ShapeDtypeStruct((N, D), jnp.float32)],
)


def _make_sc_layer(with_counts: bool):
    mesh = plsc.VectorSubcoreMesh(core_axis_name="c", subcore_axis_name="s",
                                  num_cores=NC, num_subcores=NS)
    out_type = [jax.ShapeDtypeStruct((NC, NS, RPT, 64), jnp.float32)]
    scratch = [
        pltpu.VMEM((G, CH), jnp.int32),
        pltpu.VMEM((G, CH), jnp.int32),
        pltpu.VMEM((CH, 64), jnp.float32),
        pltpu.VMEM((CH, 64), jnp.float32),
        pltpu.VMEM_SHARED((AROW, 64), jnp.float32),
        pltpu.SemaphoreType.DMA,
        pltpu.SemaphoreType.DMA,
        pltpu.SemaphoreType.DMA,
        pltpu.SemaphoreType.DMA,
    ]
    if with_counts:
        out_type.append(jax.ShapeDtypeStruct((NC, NS, RPT, 16), jnp.float32))
        scratch += [
            pltpu.VMEM((G, CH), jnp.int32),
            pltpu.VMEM((CH, 16), jnp.float32),
            pltpu.VMEM_SHARED((AROW, 16), jnp.float32),
        ]

    def body(*refs):
        if with_counts:
            (idxs, sdxh, sdxb, tab, z64, z16, onesh,
             acc_out, cnt_out,
             idx_v, sdx_v, buf0, buf1, acc_sh, sg0, sg1, ss0, ss1,
             sdxb_v, ones_v, cnt_sh) = refs
        else:
            (idxs, sdxh, tab, z64,
             acc_out,
             idx_v, sdx_v, buf0, buf1, acc_sh, sg0, sg1, ss0, ss1) = refs

        c = lax.axis_index("c")
        s = lax.axis_index("s")
        r0 = s * RPT

        pltpu.sync_copy(z64, acc_sh.at[pl.ds(r0, RPT)])
        if with_counts:
            pltpu.sync_copy(z16, cnt_sh.at[pl.ds(r0, RPT)])
            pltpu.sync_copy(onesh, ones_v)
        plsc.subcore_barrier()

        bufs = (buf0, buf1)
        sg = (sg0, sg1)
        ss = (ss0, ss1)

        def edge_super(g, carry):
            row = s * NCHK + g * G
            pltpu.sync_copy(idxs.at[c, pl.ds(row, G)], idx_v)
            pltpu.sync_copy(sdxh.at[pl.ds(row, G)], sdx_v)
            gth = [None, None]
            sct = [None, None]
            gth[0] = pltpu.async_copy(tab.at[idx_v.at[0]], bufs[0], sg[0])
            for j in range(G):
                b = j % 2
                if j + 1 < G:
                    if sct[1 - b] is not None:
                        sct[1 - b].wait()
                    gth[1 - b] = pltpu.async_copy(tab.at[idx_v.at[j + 1]],
                                                  bufs[1 - b], sg[1 - b])
                gth[b].wait()
                sct[b] = pltpu.async_copy(bufs[b], acc_sh.at[sdx_v.at[j]],
                                          ss[b], add=True)
            sct[0].wait()
            sct[1].wait()
            if with_counts:
                @pl.when((g % NC) == c)
                def _counts():
                    pltpu.sync_copy(sdxb.at[pl.ds(row, G)], sdxb_v)
                    for j in range(G):
                        pltpu.sync_copy(ones_v, cnt_sh.at[sdxb_v.at[j]],
                                        add=True)
            return carry

        lax.fori_loop(0, NSUP, edge_super, 0)

        plsc.subcore_barrier()
        pltpu.sync_copy(acc_sh.at[pl.ds(r0, RPT)], acc_out.at[c, s])
        if with_counts:
            pltpu.sync_copy(cnt_sh.at[pl.ds(r0, RPT)], cnt_out.at[c, s])

    return pl.kernel(
        body, out_type=out_type, mesh=mesh, scratch_types=scratch,
        compiler_params=pltpu.CompilerParams(use_tc_tiling_on_sc=False))


@functools.lru_cache(maxsize=None)
def _sc_layers():
    return _make_sc_layer(with_counts=True), _make_sc_layer(with_counts=False)


def kernel(des, tweet, num_prop, cat_prop, edge_index, edge_type,
           Wn, bn, Wc, bc, Wi, bi, Wrel, Wroot, brgcn, Wo1, bo1, Wo2, bo2):
    del des, tweet

    nump = jnp.pad(num_prop, ((0, 0), (0, 2)))
    catp = jnp.pad(cat_prop, ((0, 0), (0, 5)))
    wn = jnp.pad(Wn, ((0, 2), (0, 0)))
    wc = jnp.pad(Wc, ((0, 5), (0, 0)))
    wo2 = jnp.pad(Wo2, ((0, 0), (0, D - 2)))
    bo2p = jnp.pad(bo2, (0, D - 2)).reshape(1, D)
    src = edge_index[0].reshape(E2 // D, D)
    dst = edge_index[1].reshape(E2 // D, D)
    typ = edge_type.reshape(E2 // D, D)
    z64 = jnp.zeros((RPT, 64), jnp.float32)
    z16 = jnp.zeros((RPT, 16), jnp.float32)
    ones = jnp.ones((CH, 16), jnp.float32)

    idxs, sdx, sdxb = _edges(src, dst, typ)
    idxs4 = idxs.reshape(2, E2 // CH, CH)
    sdx3 = sdx.reshape(E2 // CH, CH)
    sdxb3 = sdxb.reshape(E2 // CH, CH)

    tab1, root1 = _prestage(
        nump, catp, wn, bn.reshape(1, H), wc, bc.reshape(1, H),
        Wi, bi.reshape(1, D), Wrel[0], Wrel[1], Wroot, brgcn.reshape(1, D))

    sc_layer1, sc_layer2 = _sc_layers()
    acc1, cnt = sc_layer1(idxs4, sdx3, sdxb3, tab1.reshape(4 * N, 64),
                          z64, z16, ones)
    acc1 = acc1.reshape(NC, N, D)
    cnt = cnt.reshape(NC, 2 * N, 16)

    tab2, root2 = _mid(acc1, acc1, cnt, cnt, cnt, cnt, root1,
                       Wrel[0], Wrel[1], Wroot, brgcn.reshape(1, D))

    (acc2,) = sc_layer2(idxs4, sdx3, tab2.reshape(4 * N, 64), z64)
    acc2 = acc2.reshape(NC, N, D)

    (outp,) = _head(acc2, acc2, cnt, cnt, cnt, cnt, root2,
                    Wo1, bo1.reshape(1, D), wo2, bo2p)
    return outp[:, 0:2]


# speedup vs baseline: 2.0712x; 1.0300x over previous
import functools

import jax
import jax.numpy as jnp
from jax import lax
from jax.experimental import pallas as pl
from jax.experimental.pallas import tpu as pltpu
from jax.experimental.pallas import tpu_sc as plsc

N = 10000
E = 320000
D = 128
H = 64

NC = 2
NS = 16
CH = 80
E2 = E
EPT = E2 // NS
NCHK = EPT // CH
G = 50
NSUP = NCHK // G
RPT = (2 * N) // NS
AROW = 2 * N


def _lrelu(v):
    return jnp.where(v >= 0, v, 0.01 * v)


def _dot(a, b):
    return jnp.dot(a, b, preferred_element_type=jnp.float32)


BLK = 2000
GRID = N // BLK

_row = lambda i: (i, 0)
_fix = lambda i: (0, 0)


def _edges_body(src_ref, dst_ref, typ_ref, idxs_ref, sdx_ref, sdxb_ref):
    base = 2 * (src_ref[...] + typ_ref[...] * N)
    idxs_ref[0] = base
    idxs_ref[1] = base + 1
    sdx_ref[...] = 2 * dst_ref[...] + typ_ref[...]
    sdxb_ref[...] = dst_ref[...] + typ_ref[...] * N


_edges = pl.pallas_call(
    _edges_body,
    out_shape=[
        jax.ShapeDtypeStruct((2, E2 // D, D), jnp.int32),
        jax.ShapeDtypeStruct((E2 // D, D), jnp.int32),
        jax.ShapeDtypeStruct((E2 // D, D), jnp.int32),
    ],
)


def _prestage_body(nump_ref, catp_ref, wn_ref, bn_ref, wc_ref, bc_ref,
                   wi_ref, bi_ref, wr0_ref, wr1_ref, wroot_ref, brgcn_ref,
                   tab_ref, root_ref):
    n = _lrelu(_dot(nump_ref[...], wn_ref[...]) + bn_ref[...])
    c = _lrelu(_dot(catp_ref[...], wc_ref[...]) + bc_ref[...])
    x = jnp.concatenate((n, c), axis=1)
    x = _lrelu(_dot(x, wi_ref[...]) + bi_ref[...])
    tab_ref[0] = _dot(x, wr0_ref[...])
    tab_ref[1] = _dot(x, wr1_ref[...])
    root_ref[...] = _dot(x, wroot_ref[...]) + brgcn_ref[...]


_TAB_SPEC = pl.BlockSpec((2, BLK, D), lambda i: (0, i, 0))
_TAB_OUT = jax.ShapeDtypeStruct((2, N, D), jnp.float32)
_W_SPECS = [
    pl.BlockSpec((D, D), _fix),
    pl.BlockSpec((D, D), _fix),
    pl.BlockSpec((D, D), _fix),
    pl.BlockSpec((1, D), _fix),
]

_prestage = pl.pallas_call(
    _prestage_body,
    grid=(GRID,),
    in_specs=[
        pl.BlockSpec((BLK, 8), _row),
        pl.BlockSpec((BLK, 16), _row),
        pl.BlockSpec((8, H), _fix),
        pl.BlockSpec((1, H), _fix),
        pl.BlockSpec((16, H), _fix),
        pl.BlockSpec((1, H), _fix),
        pl.BlockSpec((D, D), _fix),
        pl.BlockSpec((1, D), _fix),
    ] + _W_SPECS,
    out_specs=[_TAB_SPEC, pl.BlockSpec((BLK, D), _row)],
    out_shape=[_TAB_OUT, jax.ShapeDtypeStruct((N, D), jnp.float32)],
)


def _combine(a0, a1, c00, c10, c01, c11, root):
    agg0 = jnp.concatenate((a0[:, 0:64], a1[:, 0:64]), axis=1)
    agg1 = jnp.concatenate((a0[:, 64:128], a1[:, 64:128]), axis=1)
    inv0 = 1.0 / jnp.maximum(c00[:, 0:1] + c10[:, 0:1], 1.0)
    inv1 = 1.0 / jnp.maximum(c01[:, 0:1] + c11[:, 0:1], 1.0)
    return root + agg0 * inv0 + agg1 * inv1


_ACC_SPECS = [
    pl.BlockSpec((1, BLK, D), lambda i: (0, i, 0)),
    pl.BlockSpec((1, BLK, D), lambda i: (1, i, 0)),
    pl.BlockSpec((1, BLK, 16), lambda i: (0, i, 0)),
    pl.BlockSpec((1, BLK, 16), lambda i: (1, i, 0)),
    pl.BlockSpec((1, BLK, 16), lambda i: (0, GRID + i, 0)),
    pl.BlockSpec((1, BLK, 16), lambda i: (1, GRID + i, 0)),
    pl.BlockSpec((BLK, D), _row),
]


def _mid_body(a0_ref, a1_ref, c00_ref, c10_ref, c01_ref, c11_ref, root_ref,
              wr0_ref, wr1_ref, wroot_ref, brgcn_ref, tab_ref, root2_ref):
    x1 = _combine(a0_ref[0], a1_ref[0], c00_ref[0], c10_ref[0],
                  c01_ref[0], c11_ref[0], root_ref[...])
    tab_ref[0] = _dot(x1, wr0_ref[...])
    tab_ref[1] = _dot(x1, wr1_ref[...])
    root2_ref[...] = _dot(x1, wroot_ref[...]) + brgcn_ref[...]


_mid = pl.pallas_call(
    _mid_body,
    grid=(GRID,),
    in_specs=_ACC_SPECS + _W_SPECS,
    out_specs=[_TAB_SPEC, pl.BlockSpec((BLK, D), _row)],
    out_shape=[_TAB_OUT, jax.ShapeDtypeStruct((N, D), jnp.float32)],
)


def _head_body(a0_ref, a1_ref, c00_ref, c10_ref, c01_ref, c11_ref, root_ref,
               wo1_ref, bo1_ref, wo2_ref, bo2_ref, out_ref):
    x2 = _combine(a0_ref[0], a1_ref[0], c00_ref[0], c10_ref[0],
                  c01_ref[0], c11_ref[0], root_ref[...])
    h = _lrelu(_dot(x2, wo1_ref[...]) + bo1_ref[...])
    out_ref[...] = _dot(h, wo2_ref[...]) + bo2_ref[...]


_head = pl.pallas_call(
    _head_body,
    grid=(GRID,),
    in_specs=_ACC_SPECS + [
        pl.BlockSpec((D, D), _fix),
        pl.BlockSpec((1, D), _fix),
        pl.BlockSpec((D, D), _fix),
        pl.BlockSpec((1, D), _fix),
    ],
    out_specs=[pl.BlockSpec((BLK, D), _row)],
    out_shape=[jax.ShapeDtypeStruct((N, D), jnp.float32)],
)


def _make_sc_layer(with_counts: bool):
    mesh = plsc.VectorSubcoreMesh(core_axis_name="c", subcore_axis_name="s",
                                  num_cores=NC, num_subcores=NS)
    out_type = [jax.ShapeDtypeStruct((NC, NS, RPT, 64), jnp.float32)]
    scratch = [
        pltpu.VMEM((G, CH), jnp.int32),
        pltpu.VMEM((G, CH), jnp.int32),
        pltpu.VMEM((CH, 64), jnp.float32),
        pltpu.VMEM((CH, 64), jnp.float32),
        pltpu.VMEM_SHARED((AROW, 64), jnp.float32),
        pltpu.SemaphoreType.DMA,
        pltpu.SemaphoreType.DMA,
        pltpu.SemaphoreType.DMA,
        pltpu.SemaphoreType.DMA,
    ]
    if with_counts:
        out_type.append(jax.ShapeDtypeStruct((NC, NS, RPT, 16), jnp.float32))
        scratch += [
            pltpu.VMEM((G, CH), jnp.int32),
            pltpu.VMEM((CH, 16), jnp.float32),
            pltpu.VMEM_SHARED((AROW, 16), jnp.float32),
        ]

    def body(*refs):
        if with_counts:
            (idxs, sdxh, sdxb, tab, z64, z16, onesh,
             acc_out, cnt_out,
             idx_v, sdx_v, buf0, buf1, acc_sh, sg0, sg1, ss0, ss1,
             sdxb_v, ones_v, cnt_sh) = refs
        else:
            (idxs, sdxh, tab, z64,
             acc_out,
             idx_v, sdx_v, buf0, buf1, acc_sh, sg0, sg1, ss0, ss1) = refs

        c = lax.axis_index("c")
        s = lax.axis_index("s")
        r0 = s * RPT

        pltpu.sync_copy(z64, acc_sh.at[pl.ds(r0, RPT)])
        if with_counts:
            pltpu.sync_copy(z16, cnt_sh.at[pl.ds(r0, RPT)])
            pltpu.sync_copy(onesh, ones_v)
        plsc.subcore_barrier()

        bufs = (buf0, buf1)
        sg = (sg0, sg1)
        ss = (ss0, ss1)

        def edge_super(g, carry):
            row = s * NCHK + g * G
            pltpu.sync_copy(idxs.at[c, pl.ds(row, G)], idx_v)
            pltpu.sync_copy(sdxh.at[pl.ds(row, G)], sdx_v)
            gth = [None, None]
            sct = [None, None]
            gth[0] = pltpu.async_copy(tab.at[idx_v.at[0]], bufs[0], sg[0])
            for j in range(G):
                b = j % 2
                if j + 1 < G:
                    if sct[1 - b] is not None:
                        sct[1 - b].wait()
                    gth[1 - b] = pltpu.async_copy(tab.at[idx_v.at[j + 1]],
                                                  bufs[1 - b], sg[1 - b])
                gth[b].wait()
                sct[b] = pltpu.async_copy(bufs[b], acc_sh.at[sdx_v.at[j]],
                                          ss[b], add=True)
            sct[0].wait()
            sct[1].wait()
            if with_counts:
                @pl.when((g % NC) == c)
                def _counts():
                    pltpu.sync_copy(sdxb.at[pl.ds(row, G)], sdxb_v)
                    for j in range(G):
                        pltpu.sync_copy(ones_v, cnt_sh.at[sdxb_v.at[j]],
                                        add=True)
            return carry

        lax.fori_loop(0, NSUP, edge_super, 0)

        plsc.subcore_barrier()
        pltpu.sync_copy(acc_sh.at[pl.ds(r0, RPT)], acc_out.at[c, s])
        if with_counts:
            pltpu.sync_copy(cnt_sh.at[pl.ds(r0, RPT)], cnt_out.at[c, s])

    return pl.kernel(
        body, out_type=out_type, mesh=mesh, scratch_types=scratch,
        compiler_params=pltpu.CompilerParams(use_tc_tiling_on_sc=False))


@functools.lru_cache(maxsize=None)
def _sc_layers():
    return _make_sc_layer(with_counts=True), _make_sc_layer(with_counts=False)


def kernel(des, tweet, num_prop, cat_prop, edge_index, edge_type,
           Wn, bn, Wc, bc, Wi, bi, Wrel, Wroot, brgcn, Wo1, bo1, Wo2, bo2):
    del des, tweet

    nump = jnp.pad(num_prop, ((0, 0), (0, 2)))
    catp = jnp.pad(cat_prop, ((0, 0), (0, 5)))
    wn = jnp.pad(Wn, ((0, 2), (0, 0)))
    wc = jnp.pad(Wc, ((0, 5), (0, 0)))
    wo2 = jnp.pad(Wo2, ((0, 0), (0, D - 2)))
    bo2p = jnp.pad(bo2, (0, D - 2)).reshape(1, D)
    src = edge_index[0].reshape(E2 // D, D)
    dst = edge_index[1].reshape(E2 // D, D)
    typ = edge_type.reshape(E2 // D, D)
    z64 = jnp.zeros((RPT, 64), jnp.float32)
    z16 = jnp.zeros((RPT, 16), jnp.float32)
    ones = jnp.ones((CH, 16), jnp.float32)

    idxs, sdx, sdxb = _edges(src, dst, typ)
    idxs4 = idxs.reshape(2, E2 // CH, CH)
    sdx3 = sdx.reshape(E2 // CH, CH)
    sdxb3 = sdxb.reshape(E2 // CH, CH)

    tab1, root1 = _prestage(
        nump, catp, wn, bn.reshape(1, H), wc, bc.reshape(1, H),
        Wi, bi.reshape(1, D), Wrel[0], Wrel[1], Wroot, brgcn.reshape(1, D))

    sc_layer1, sc_layer2 = _sc_layers()
    acc1, cnt = sc_layer1(idxs4, sdx3, sdxb3, tab1.reshape(4 * N, 64),
                          z64, z16, ones)
    acc1 = acc1.reshape(NC, N, D)
    cnt = cnt.reshape(NC, 2 * N, 16)

    tab2, root2 = _mid(acc1, acc1, cnt, cnt, cnt, cnt, root1,
                       Wrel[0], Wrel[1], Wroot, brgcn.reshape(1, D))

    (acc2,) = sc_layer2(idxs4, sdx3, tab2.reshape(4 * N, 64), z64)
    acc2 = acc2.reshape(NC, N, D)

    (outp,) = _head(acc2, acc2, cnt, cnt, cnt, cnt, root2,
                    Wo1, bo1.reshape(1, D), wo2, bo2p)
    return outp[:, 0:2]


# speedup vs baseline: 2.1902x; 1.0575x over previous
import functools

import jax
import jax.numpy as jnp
from jax import lax
from jax.experimental import pallas as pl
from jax.experimental.pallas import tpu as pltpu
from jax.experimental.pallas import tpu_sc as plsc

N = 10000
E = 320000
D = 128
H = 64

NC = 2
NS = 16
CH = 80
E2 = E
EPT = E2 // NS
NCHK = EPT // CH
G = 50
NSUP = NCHK // G
RPT = (2 * N) // NS
AROW = 2 * N


def _lrelu(v):
    return jnp.where(v >= 0, v, 0.01 * v)


def _dot(a, b):
    return jnp.dot(a, b, preferred_element_type=jnp.float32)


BLK = 2000
GRID = N // BLK

_row = lambda i: (i, 0)
_fix = lambda i: (0, 0)


def _edges_body(src_ref, dst_ref, typ_ref, idxs_ref, sdx_ref):
    base = 2 * (src_ref[...] + typ_ref[...] * N)
    idxs_ref[0] = base
    idxs_ref[1] = base + 1
    sdx_ref[...] = 2 * dst_ref[...] + typ_ref[...]


_edges = pl.pallas_call(
    _edges_body,
    out_shape=[
        jax.ShapeDtypeStruct((2, E2 // D, D), jnp.int32),
        jax.ShapeDtypeStruct((E2 // D, D), jnp.int32),
    ],
)


def _prestage_body(nump_ref, catp_ref, wn_ref, bn_ref, wc_ref, bc_ref,
                   wi_ref, bi_ref, wr0_ref, wr1_ref, wroot_ref, brgcn_ref,
                   tab_ref, root_ref):
    n = _lrelu(_dot(nump_ref[...], wn_ref[...]) + bn_ref[...])
    c = _lrelu(_dot(catp_ref[...], wc_ref[...]) + bc_ref[...])
    x = jnp.concatenate((n, c), axis=1)
    x = _lrelu(_dot(x, wi_ref[...]) + bi_ref[...])
    tab_ref[0] = _dot(x, wr0_ref[...])
    tab_ref[1] = _dot(x, wr1_ref[...])
    root_ref[...] = _dot(x, wroot_ref[...]) + brgcn_ref[...]


_TAB_SPEC = pl.BlockSpec((2, BLK, D), lambda i: (0, i, 0))
_TAB_OUT = jax.ShapeDtypeStruct((2, N, D), jnp.float32)
_W_SPECS = [
    pl.BlockSpec((D, D), _fix),
    pl.BlockSpec((D, D), _fix),
    pl.BlockSpec((D, D), _fix),
    pl.BlockSpec((1, D), _fix),
]

_prestage = pl.pallas_call(
    _prestage_body,
    grid=(GRID,),
    in_specs=[
        pl.BlockSpec((BLK, 8), _row),
        pl.BlockSpec((BLK, 16), _row),
        pl.BlockSpec((8, H), _fix),
        pl.BlockSpec((1, H), _fix),
        pl.BlockSpec((16, H), _fix),
        pl.BlockSpec((1, H), _fix),
        pl.BlockSpec((D, D), _fix),
        pl.BlockSpec((1, D), _fix),
    ] + _W_SPECS,
    out_specs=[_TAB_SPEC, pl.BlockSpec((BLK, D), _row)],
    out_shape=[_TAB_OUT, jax.ShapeDtypeStruct((N, D), jnp.float32)],
)


def _combine(a0, a1, c0, c1, root):
    agg0 = jnp.concatenate((a0[:, 0:64], a1[:, 0:64]), axis=1)
    agg1 = jnp.concatenate((a0[:, 64:128], a1[:, 64:128]), axis=1)
    inv0 = 1.0 / jnp.maximum(c0[:, 0:1] + c1[:, 0:1], 1.0)
    inv1 = 1.0 / jnp.maximum(c0[:, 16:17] + c1[:, 16:17], 1.0)
    return root + agg0 * inv0 + agg1 * inv1


_ACC_SPECS = [
    pl.BlockSpec((1, BLK, D), lambda i: (0, i, 0)),
    pl.BlockSpec((1, BLK, D), lambda i: (1, i, 0)),
    pl.BlockSpec((1, BLK, 32), lambda i: (0, i, 0)),
    pl.BlockSpec((1, BLK, 32), lambda i: (1, i, 0)),
    pl.BlockSpec((BLK, D), _row),
]


def _mid_body(a0_ref, a1_ref, c0_ref, c1_ref, root_ref,
              wr0_ref, wr1_ref, wroot_ref, brgcn_ref, tab_ref, root2_ref):
    x1 = _combine(a0_ref[0], a1_ref[0], c0_ref[0], c1_ref[0], root_ref[...])
    tab_ref[0] = _dot(x1, wr0_ref[...])
    tab_ref[1] = _dot(x1, wr1_ref[...])
    root2_ref[...] = _dot(x1, wroot_ref[...]) + brgcn_ref[...]


_mid = pl.pallas_call(
    _mid_body,
    grid=(GRID,),
    in_specs=_ACC_SPECS + _W_SPECS,
    out_specs=[_TAB_SPEC, pl.BlockSpec((BLK, D), _row)],
    out_shape=[_TAB_OUT, jax.ShapeDtypeStruct((N, D), jnp.float32)],
)


def _head_body(a0_ref, a1_ref, c0_ref, c1_ref, root_ref,
               wo1_ref, bo1_ref, wo2_ref, bo2_ref, out_ref):
    x2 = _combine(a0_ref[0], a1_ref[0], c0_ref[0], c1_ref[0], root_ref[...])
    h = _lrelu(_dot(x2, wo1_ref[...]) + bo1_ref[...])
    out_ref[...] = _dot(h, wo2_ref[...]) + bo2_ref[...]


_head = pl.pallas_call(
    _head_body,
    grid=(GRID,),
    in_specs=_ACC_SPECS + [
        pl.BlockSpec((D, D), _fix),
        pl.BlockSpec((1, D), _fix),
        pl.BlockSpec((D, D), _fix),
        pl.BlockSpec((1, D), _fix),
    ],
    out_specs=[pl.BlockSpec((BLK, D), _row)],
    out_shape=[jax.ShapeDtypeStruct((N, D), jnp.float32)],
)


def _make_sc_layer(with_counts: bool):
    mesh = plsc.VectorSubcoreMesh(core_axis_name="c", subcore_axis_name="s",
                                  num_cores=NC, num_subcores=NS)
    out_type = [jax.ShapeDtypeStruct((NC, NS, RPT, 64), jnp.float32)]
    scratch = [
        pltpu.VMEM((G, CH), jnp.int32),
        pltpu.VMEM((G, CH), jnp.int32),
        pltpu.VMEM((CH, 64), jnp.float32),
        pltpu.VMEM((CH, 64), jnp.float32),
        pltpu.VMEM_SHARED((AROW, 64), jnp.float32),
        pltpu.SemaphoreType.DMA,
        pltpu.SemaphoreType.DMA,
        pltpu.SemaphoreType.DMA,
        pltpu.SemaphoreType.DMA,
    ]
    if with_counts:
        out_type.append(jax.ShapeDtypeStruct((NC, NS, RPT, 16), jnp.float32))
        scratch += [
            pltpu.SemaphoreType.DMA,
            pltpu.VMEM((CH, 16), jnp.float32),
            pltpu.VMEM_SHARED((AROW, 16), jnp.float32),
        ]

    def body(*refs):
        if with_counts:
            (idxs, sdxh, tab, z64, z16, onesh,
             acc_out, cnt_out,
             idx_v, sdx_v, buf0, buf1, acc_sh, sg0, sg1, ss0, ss1,
             sc_, ones_v, cnt_sh) = refs
        else:
            (idxs, sdxh, tab, z64,
             acc_out,
             idx_v, sdx_v, buf0, buf1, acc_sh, sg0, sg1, ss0, ss1) = refs

        c = lax.axis_index("c")
        s = lax.axis_index("s")
        r0 = s * RPT

        pltpu.sync_copy(z64, acc_sh.at[pl.ds(r0, RPT)])
        if with_counts:
            pltpu.sync_copy(z16, cnt_sh.at[pl.ds(r0, RPT)])
            pltpu.sync_copy(onesh, ones_v)
        plsc.subcore_barrier()

        bufs = (buf0, buf1)
        sg = (sg0, sg1)
        ss = (ss0, ss1)

        def edge_super(g, carry):
            row = s * NCHK + g * G
            pltpu.sync_copy(idxs.at[c, pl.ds(row, G)], idx_v)
            pltpu.sync_copy(sdxh.at[pl.ds(row, G)], sdx_v)
            gth = [None, None]
            sct = [None, None]
            gth[0] = pltpu.async_copy(tab.at[idx_v.at[0]], bufs[0], sg[0])
            for j in range(G):
                b = j % 2
                if j + 1 < G:
                    if sct[1 - b] is not None:
                        sct[1 - b].wait()
                    gth[1 - b] = pltpu.async_copy(tab.at[idx_v.at[j + 1]],
                                                  bufs[1 - b], sg[1 - b])
                gth[b].wait()
                sct[b] = pltpu.async_copy(bufs[b], acc_sh.at[sdx_v.at[j]],
                                          ss[b], add=True)
                if with_counts:
                    @pl.when((g % NC) == c)
                    def _count_j():
                        pltpu.async_copy(ones_v, cnt_sh.at[sdx_v.at[j]],
                                         sc_, add=True)
            sct[0].wait()
            sct[1].wait()
            if with_counts:
                @pl.when((g % NC) == c)
                def _count_drain():
                    for _ in range(G):
                        pltpu.make_async_copy(ones_v, cnt_sh.at[sdx_v.at[0]],
                                              sc_).wait()
            return carry

        lax.fori_loop(0, NSUP, edge_super, 0)

        plsc.subcore_barrier()
        pltpu.sync_copy(acc_sh.at[pl.ds(r0, RPT)], acc_out.at[c, s])
        if with_counts:
            pltpu.sync_copy(cnt_sh.at[pl.ds(r0, RPT)], cnt_out.at[c, s])

    return pl.kernel(
        body, out_type=out_type, mesh=mesh, scratch_types=scratch,
        compiler_params=pltpu.CompilerParams(use_tc_tiling_on_sc=False))


@functools.lru_cache(maxsize=None)
def _sc_layers():
    return _make_sc_layer(with_counts=True), _make_sc_layer(with_counts=False)


def kernel(des, tweet, num_prop, cat_prop, edge_index, edge_type,
           Wn, bn, Wc, bc, Wi, bi, Wrel, Wroot, brgcn, Wo1, bo1, Wo2, bo2):
    del des, tweet

    nump = jnp.pad(num_prop, ((0, 0), (0, 2)))
    catp = jnp.pad(cat_prop, ((0, 0), (0, 5)))
    wn = jnp.pad(Wn, ((0, 2), (0, 0)))
    wc = jnp.pad(Wc, ((0, 5), (0, 0)))
    wo2 = jnp.pad(Wo2, ((0, 0), (0, D - 2)))
    bo2p = jnp.pad(bo2, (0, D - 2)).reshape(1, D)
    src = edge_index[0].reshape(E2 // D, D)
    dst = edge_index[1].reshape(E2 // D, D)
    typ = edge_type.reshape(E2 // D, D)
    z64 = jnp.zeros((RPT, 64), jnp.float32)
    z16 = jnp.zeros((RPT, 16), jnp.float32)
    ones = jnp.ones((CH, 16), jnp.float32)

    idxs, sdx = _edges(src, dst, typ)
    idxs4 = idxs.reshape(2, E2 // CH, CH)
    sdx3 = sdx.reshape(E2 // CH, CH)

    tab1, root1 = _prestage(
        nump, catp, wn, bn.reshape(1, H), wc, bc.reshape(1, H),
        Wi, bi.reshape(1, D), Wrel[0], Wrel[1], Wroot, brgcn.reshape(1, D))

    sc_layer1, sc_layer2 = _sc_layers()
    acc1, cnt = sc_layer1(idxs4, sdx3, tab1.reshape(4 * N, 64),
                          z64, z16, ones)
    acc1 = acc1.reshape(NC, N, D)
    cnt = cnt.reshape(NC, N, 32)

    tab2, root2 = _mid(acc1, acc1, cnt, cnt, root1,
                       Wrel[0], Wrel[1], Wroot, brgcn.reshape(1, D))

    (acc2,) = sc_layer2(idxs4, sdx3, tab2.reshape(4 * N, 64), z64)
    acc2 = acc2.reshape(NC, N, D)

    (outp,) = _head(acc2, acc2, cnt, cnt, root2,
                    Wo1, bo1.reshape(1, D), wo2, bo2p)
    return outp[:, 0:2]
